# baseline (speedup 1.0000x reference)
import functools

import jax
import jax.numpy as jnp
from jax import lax
from jax.experimental import pallas as pl
from jax.experimental.pallas import tpu as pltpu
from jax.experimental.pallas import tpu_sc as plsc

DIM = 300
SEQ = 200
TOK = 180
BATCH = 1024
TILEW = 128
SEG = 3
TPAD = 184
IPAD = 192
SROWS = SEG * TPAD
HA, HB = 96, 88


@functools.lru_cache(maxsize=1)
def _make_sc_gather():
    info = plsc.get_sparse_core_info()
    nw = info.num_cores * info.num_subcores
    bpw = BATCH // nw
    mesh = plsc.VectorSubcoreMesh(core_axis_name="c", subcore_axis_name="s")

    @functools.partial(
        pl.kernel,
        mesh=mesh,
        out_type=jax.ShapeDtypeStruct((BATCH * SROWS, TILEW), jnp.float32),
        scratch_types=[
            pltpu.VMEM((IPAD,), jnp.int32),
            pltpu.VMEM((SROWS, TILEW), jnp.float32),
            pltpu.SemaphoreType.DMA,
            pltpu.SemaphoreType.DMA,
        ],
    )
    def gather_kernel(idx_hbm, table_hbm, tail_hbm, out_hbm,
                      iv, pkd, gsem, ssem):
        wid = lax.axis_index("s") * info.num_cores + lax.axis_index("c")
        b0 = wid * bpw
        views = (
            table_hbm.at[:, pl.ds(0, TILEW)],
            table_hbm.at[:, pl.ds(TILEW, TILEW)],
            tail_hbm,
        )

        def body(j, carry):
            pltpu.sync_copy(idx_hbm.at[pl.ds(IPAD * (b0 + j), IPAD)], iv)

            @pl.when(j >= 1)
            def _():
                pltpu.make_async_copy(
                    pkd, out_hbm.at[pl.ds(SROWS * b0, SROWS)], ssem).wait()

            descs = []
            for ct in range(SEG):
                for off, n in ((0, HA), (HA, HB)):
                    descs.append(pltpu.async_copy(
                        views[ct].at[iv.at[pl.ds(off, n)]],
                        pkd.at[pl.ds(TPAD * ct + off, n)], gsem))
            for d in descs:
                d.wait()
            pltpu.async_copy(
                pkd, out_hbm.at[pl.ds(SROWS * (b0 + j), SROWS)], ssem)
            return carry

        lax.fori_loop(0, bpw, body, 0)
        pltpu.make_async_copy(
            pkd, out_hbm.at[pl.ds(SROWS * b0, SROWS)], ssem).wait()

    return gather_kernel


def _assemble_body(in_ref, out_ref):
    a = in_ref[0, 0:TOK, :]
    b = in_ref[0, TPAD:TPAD + TOK, :]
    c = in_ref[0, 2 * TPAD:2 * TPAD + TOK, TILEW - (DIM - 2 * TILEW):]
    row = jnp.concatenate([a, b, c], axis=1)
    z = jnp.zeros((SEQ - TOK, DIM), jnp.float32)
    out_ref[0] = jnp.concatenate([row, z], axis=0)


@functools.lru_cache(maxsize=1)
def _make_assemble():
    return pl.pallas_call(
        _assemble_body,
        grid=(BATCH,),
        in_specs=[pl.BlockSpec((1, SROWS, TILEW), lambda b: (b, 0, 0))],
        out_specs=pl.BlockSpec((1, SEQ, DIM), lambda b: (b, 0, 0)),
        out_shape=jax.ShapeDtypeStruct((BATCH, SEQ, DIM), jnp.float32),
    )


def kernel(indices, table):
    idx = jnp.pad(indices, ((0, 0), (0, IPAD - TOK))).reshape(-1)
    tail = table[:, DIM - TILEW:]
    out = _make_sc_gather()(idx, table, tail)
    o = out.reshape(BATCH, SROWS, TILEW)
    return _make_assemble()(o)


# speedup vs baseline: 1.1422x; 1.1422x over previous
import functools

import jax
import jax.numpy as jnp
from jax import lax
from jax.experimental import pallas as pl
from jax.experimental.pallas import tpu as pltpu
from jax.experimental.pallas import tpu_sc as plsc

DIM = 300
SEQ = 200
TOK = 180
BATCH = 1024
TILEW = 128
SEG = 3
TPAD = 184
IPAD = 192
SROWS = SEG * TPAD
HA, HB = 96, 88


@functools.lru_cache(maxsize=1)
def _make_sc_gather():
    info = plsc.get_sparse_core_info()
    nw = info.num_cores * info.num_subcores
    bpw = BATCH // nw
    mesh = plsc.VectorSubcoreMesh(core_axis_name="c", subcore_axis_name="s")

    @functools.partial(
        pl.kernel,
        mesh=mesh,
        out_type=jax.ShapeDtypeStruct((BATCH * SROWS, TILEW), jnp.float32),
        scratch_types=[
            pltpu.VMEM((IPAD,), jnp.int32),
            pltpu.VMEM((SROWS, TILEW), jnp.float32),
            pltpu.SemaphoreType.DMA,
            pltpu.SemaphoreType.DMA,
        ],
    )
    def gather_kernel(idx_hbm, table_hbm, tail_hbm, out_hbm,
                      iv, pkd, gsem, ssem):
        wid = lax.axis_index("s") * info.num_cores + lax.axis_index("c")
        b0 = wid * bpw
        views = (
            table_hbm.at[:, pl.ds(0, TILEW)],
            table_hbm.at[:, pl.ds(TILEW, TILEW)],
            tail_hbm,
        )

        def body(j, carry):
            pltpu.sync_copy(idx_hbm.at[pl.ds(IPAD * (b0 + j), IPAD)], iv)

            @pl.when(j >= 1)
            def _():
                pltpu.make_async_copy(
                    pkd, out_hbm.at[pl.ds(SROWS * b0, SROWS)], ssem).wait()

            descs = []
            for ct in range(SEG):
                for off, n in ((0, HA), (HA, HB)):
                    descs.append(pltpu.async_copy(
                        views[ct].at[iv.at[pl.ds(off, n)]],
                        pkd.at[pl.ds(TPAD * ct + off, n)], gsem))
            for d in descs:
                d.wait()
            pltpu.async_copy(
                pkd, out_hbm.at[pl.ds(SROWS * (b0 + j), SROWS)], ssem)
            return carry

        lax.fori_loop(0, bpw, body, 0)
        pltpu.make_async_copy(
            pkd, out_hbm.at[pl.ds(SROWS * b0, SROWS)], ssem).wait()

    return gather_kernel


def kernel(indices, table):
    idx = jnp.pad(indices, ((0, 0), (0, IPAD - TOK))).reshape(-1)
    tail = table[:, DIM - TILEW:]
    out = _make_sc_gather()(idx, table, tail)
    o = out.reshape(BATCH, SEG, TPAD, TILEW)
    emb = jnp.concatenate(
        [o[:, 0, :TOK], o[:, 1, :TOK],
         o[:, 2, :TOK, TILEW - (DIM - 2 * TILEW):]], axis=-1)
    return jnp.pad(emb, ((0, 0), (0, SEQ - TOK), (0, 0)))
